# Initial kernel scaffold; baseline (speedup 1.0000x reference)
import jax
import jax.numpy as jnp
from jax.experimental import pallas as pl


def kernel(points, xyz1, xyz2):
    raise NotImplementedError("write your pallas kernel here")


# speedup vs baseline: 36.2640x; 36.2640x over previous
import functools

import jax
import jax.numpy as jnp
from jax.experimental import pallas as pl


def _interp_kernel(xyz1t_ref, xyz2t_ref, points_ref, out_ref, *, M):
    qx = xyz1t_ref[0, 0, :][:, None]
    qy = xyz1t_ref[0, 1, :][:, None]
    qz = xyz1t_ref[0, 2, :][:, None]
    px = xyz2t_ref[0, 0, :][None, :]
    py = xyz2t_ref[0, 1, :][None, :]
    pz = xyz2t_ref[0, 2, :][None, :]

    dx = qx - px
    dy = qy - py
    dz = qz - pz
    d2 = dx * dx + dy * dy + dz * dz

    iota = jax.lax.broadcasted_iota(jnp.int32, d2.shape, 1)

    work = d2
    vals = []
    idxs = []
    for _ in range(3):
        v = jnp.min(work, axis=1, keepdims=True)
        hit = work == v
        idx = jnp.min(jnp.where(hit, iota, M), axis=1, keepdims=True)
        vals.append(v)
        idxs.append(idx)
        work = jnp.where(iota == idx, jnp.inf, work)

    inv = [1.0 / jnp.maximum(v, 1e-10) for v in vals]
    norm = inv[0] + inv[1] + inv[2]
    w = [iv / norm for iv in inv]

    W = (
        jnp.where(iota == idxs[0], w[0], 0.0)
        + jnp.where(iota == idxs[1], w[1], 0.0)
        + jnp.where(iota == idxs[2], w[2], 0.0)
    )

    out_ref[0, :, :] = jnp.dot(
        W, points_ref[0, :, :], preferred_element_type=jnp.float32
    )


@functools.partial(jax.jit, static_argnames=("tile_n",))
def _run(points, xyz1, xyz2, tile_n=512):
    B, N, _ = xyz1.shape
    _, M, C = points.shape
    xyz1t = jnp.transpose(xyz1, (0, 2, 1))
    xyz2t = jnp.transpose(xyz2, (0, 2, 1))

    grid = (B, N // tile_n)
    return pl.pallas_call(
        functools.partial(_interp_kernel, M=M),
        grid=grid,
        in_specs=[
            pl.BlockSpec((1, 3, tile_n), lambda b, n: (b, 0, n)),
            pl.BlockSpec((1, 3, M), lambda b, n: (b, 0, 0)),
            pl.BlockSpec((1, M, C), lambda b, n: (b, 0, 0)),
        ],
        out_specs=pl.BlockSpec((1, tile_n, C), lambda b, n: (b, n, 0)),
        out_shape=jax.ShapeDtypeStruct((B, N, C), jnp.float32),
    )(xyz1t, xyz2t, points)


def kernel(points, xyz1, xyz2):
    return _run(points, xyz1, xyz2)


# speedup vs baseline: 57.7155x; 1.5915x over previous
import functools

import jax
import jax.numpy as jnp
from jax.experimental import pallas as pl


def _interp_kernel(xyz1t_ref, xyz2t_ref, points_ref, out_ref, *, M):
    qx = xyz1t_ref[0, 0, :][:, None]
    qy = xyz1t_ref[0, 1, :][:, None]
    qz = xyz1t_ref[0, 2, :][:, None]
    px = xyz2t_ref[0, 0, :][None, :]
    py = xyz2t_ref[0, 1, :][None, :]
    pz = xyz2t_ref[0, 2, :][None, :]

    dx = qx - px
    dy = qy - py
    dz = qz - pz
    d2 = dx * dx + dy * dy + dz * dz

    v1 = jnp.min(d2, axis=1, keepdims=True)
    t = jnp.where(d2 > v1, d2, jnp.inf)
    v2 = jnp.min(t, axis=1, keepdims=True)
    t = jnp.where(t > v2, t, jnp.inf)
    v3 = jnp.min(t, axis=1, keepdims=True)

    inv = 1.0 / jnp.maximum(d2, 1e-10)
    masked = jnp.where(d2 <= v3, inv, 0.0)
    norm = jnp.sum(masked, axis=1, keepdims=True)
    W = masked * (1.0 / norm)

    out_ref[0, :, :] = jnp.dot(
        W, points_ref[0, :, :], preferred_element_type=jnp.float32
    )


@functools.partial(jax.jit, static_argnames=("tile_n",))
def _run(points, xyz1, xyz2, tile_n=512):
    B, N, _ = xyz1.shape
    _, M, C = points.shape
    xyz1t = jnp.transpose(xyz1, (0, 2, 1))
    xyz2t = jnp.transpose(xyz2, (0, 2, 1))

    grid = (B, N // tile_n)
    return pl.pallas_call(
        functools.partial(_interp_kernel, M=M),
        grid=grid,
        in_specs=[
            pl.BlockSpec((1, 3, tile_n), lambda b, n: (b, 0, n)),
            pl.BlockSpec((1, 3, M), lambda b, n: (b, 0, 0)),
            pl.BlockSpec((1, M, C), lambda b, n: (b, 0, 0)),
        ],
        out_specs=pl.BlockSpec((1, tile_n, C), lambda b, n: (b, n, 0)),
        out_shape=jax.ShapeDtypeStruct((B, N, C), jnp.float32),
    )(xyz1t, xyz2t, points)


def kernel(points, xyz1, xyz2):
    return _run(points, xyz1, xyz2)
